# baseline (speedup 1.0000x reference)
import functools

import jax
import jax.numpy as jnp
from jax import lax
from jax.experimental import pallas as pl
from jax.experimental.pallas import tpu as pltpu
from jax.experimental.pallas import tpu_sc as plsc

N = 10000
NPAD = 10240
E = 320000
D_IN = 128
D_HID = 128
D_OUT = 64
NC = 2
NS = 16
NW = NC * NS
EW = E // NW
CH = 80
NB = EW // CH
CHP = 40
NBP = EW // CHP
NBB = 25
IB = NBP // NBB
RPT = NPAD // NS
BR = 1024
GRID = NPAD // BR

_mesh = plsc.VectorSubcoreMesh(core_axis_name="c", subcore_axis_name="s")


@functools.partial(
    pl.kernel,
    out_type=jax.ShapeDtypeStruct((NC, 2, NPAD), jnp.float32),
    mesh=_mesh,
    scratch_types=[
        pltpu.VMEM((NBB, CHP), jnp.int32),
        pltpu.VMEM((NBB, CHP), jnp.int32),
        pltpu.VMEM((CHP,), jnp.float32),
        pltpu.VMEM((RPT,), jnp.float32),
        pltpu.VMEM_SHARED((NPAD,), jnp.float32),
        pltpu.VMEM_SHARED((NPAD,), jnp.float32),
        pltpu.SemaphoreType.DMA,
        pltpu.SemaphoreType.DMA,
    ],
)
def _degrees(src_hbm, dst_hbm, out_hbm, sidx, didx, ones_v, zer_v, acc_s, acc_d,
             ssem, dsem):
    c = lax.axis_index("c")
    s = lax.axis_index("s")
    wid = c * NS + s
    for i in range(CHP // 16):
        ones_v[pl.ds(i * 16, 16)] = jnp.ones((16,), jnp.float32)
    if CHP % 16:
        ones_v[pl.ds(CHP - 16, 16)] = jnp.ones((16,), jnp.float32)
    for i in range(RPT // 16):
        zer_v[pl.ds(i * 16, 16)] = jnp.zeros((16,), jnp.float32)
    r0 = pl.multiple_of(s * RPT, 128)
    pltpu.sync_copy(zer_v, acc_s.at[pl.ds(r0, RPT)])
    pltpu.sync_copy(zer_v, acc_d.at[pl.ds(r0, RPT)])
    plsc.subcore_barrier()

    def blk_body(blk, carry):
        pltpu.sync_copy(src_hbm.at[wid, blk], sidx)
        pltpu.sync_copy(dst_hbm.at[wid, blk], didx)

        def body(g, carry2):
            ds_ = []
            for j in range(5):
                b = g * 5 + j
                ds_.append(
                    pltpu.async_copy(ones_v, acc_s.at[sidx.at[b]], dsem, add=True)
                )
                ds_.append(
                    pltpu.async_copy(ones_v, acc_d.at[didx.at[b]], ssem, add=True)
                )
            for d in ds_:
                d.wait()
            return carry2

        lax.fori_loop(0, NBB // 5, body, 0)
        return carry

    lax.fori_loop(0, IB, blk_body, 0)
    plsc.subcore_barrier()
    pltpu.sync_copy(acc_s.at[pl.ds(r0, RPT)], out_hbm.at[c, 0, pl.ds(r0, RPT)])
    pltpu.sync_copy(acc_d.at[pl.ds(r0, RPT)], out_hbm.at[c, 1, pl.ds(r0, RPT)])


NBUF = 5


def _make_prop(DA):
    @functools.partial(
        pl.kernel,
        out_type=jax.ShapeDtypeStruct((NC, NPAD, DA), jnp.float32),
        mesh=_mesh,
        scratch_types=[
            pltpu.VMEM((2, NBB, CHP), jnp.int32),
            pltpu.VMEM((2, NBB, CHP), jnp.int32),
            pltpu.VMEM((NBUF, CHP, D_HID), jnp.float32),
            pltpu.VMEM((16, DA), jnp.float32),
            pltpu.VMEM_SHARED((NPAD, DA), jnp.float32),
        ]
        + [pltpu.SemaphoreType.DMA] * (2 * NBUF + 2),
    )
    def _prop(src_hbm, dst_hbm, feat_hbm, out_hbm, sidx, didx, rows_v, z16, acc, *sems):
        gsems = sems[:NBUF]
        ssems = sems[NBUF : 2 * NBUF]
        isems = sems[2 * NBUF :]
        c = lax.axis_index("c")
        s = lax.axis_index("s")
        wid = c * NS + s
        ids = [None, None]
        ids[0] = (
            pltpu.async_copy(src_hbm.at[wid, 0], sidx.at[0], isems[0]),
            pltpu.async_copy(dst_hbm.at[wid, 0], didx.at[0], isems[1]),
        )
        for r in range(16):
            for j in range(DA // 16):
                z16[r, pl.ds(j * 16, 16)] = jnp.zeros((16,), jnp.float32)
        r0 = pl.multiple_of(s * RPT, 128)
        zds = [
            pltpu.async_copy(z16, acc.at[pl.ds(r0 + k * 16, 16)], gsems[k % NBUF])
            for k in range(RPT // 16)
        ]
        for d in zds:
            d.wait()
        plsc.subcore_barrier()

        for blk in range(IB):
            sl = blk % 2
            for d in ids[sl]:
                d.wait()
            if blk + 1 < IB:
                nsl = (blk + 1) % 2
                ids[nsl] = (
                    pltpu.async_copy(src_hbm.at[wid, blk + 1], sidx.at[nsl], isems[0]),
                    pltpu.async_copy(dst_hbm.at[wid, blk + 1], didx.at[nsl], isems[1]),
                )
            pending = [None] * NBUF
            for g in range(NBB // NBUF):
                gds = []
                for j in range(NBUF):
                    if pending[j] is not None:
                        pending[j].wait()
                    gds.append(
                        pltpu.async_copy(
                            feat_hbm.at[sidx.at[sl, g * NBUF + j]],
                            rows_v.at[j],
                            gsems[j],
                        )
                    )
                for j in range(NBUF):
                    gds[j].wait()
                    pending[j] = pltpu.async_copy(
                        rows_v.at[j],
                        acc.at[didx.at[sl, g * NBUF + j]],
                        ssems[j],
                        add=True,
                    )
            for j in range(NBUF):
                pending[j].wait()
        plsc.subcore_barrier()
        pltpu.sync_copy(acc.at[pl.ds(r0, RPT)], out_hbm.at[c, pl.ds(r0, RPT)])

    return _prop


_prop128 = _make_prop(D_HID)
_prop64 = _prop128


def _mm1_body(x_ref, w_ref, deg_ref, o_ref):
    h = jnp.dot(x_ref[...], w_ref[...], preferred_element_type=jnp.float32)
    ns = lax.rsqrt(jnp.maximum(deg_ref[0, 0] + deg_ref[1, 0], 1.0))
    o_ref[...] = h * ns[:, None]


def _mm1(x, W_gcn, degp):
    return pl.pallas_call(
        _mm1_body,
        grid=(GRID,),
        in_specs=[
            pl.BlockSpec((BR, D_IN), lambda i: (i, 0)),
            pl.BlockSpec((D_IN, D_HID), lambda i: (0, 0)),
            pl.BlockSpec((NC, 2, BR), lambda i: (0, 0, i)),
        ],
        out_specs=pl.BlockSpec((BR, D_HID), lambda i: (i, 0)),
        out_shape=jax.ShapeDtypeStruct((N, D_HID), jnp.float32),
    )(x, W_gcn, degp)


def _mid_body(p_ref, deg_ref, w_ref, hcat_ref):
    nd = lax.rsqrt(jnp.maximum(deg_ref[0, 1] + deg_ref[1, 1], 1.0))
    ns = lax.rsqrt(jnp.maximum(deg_ref[0, 0] + deg_ref[1, 0], 1.0))
    s = p_ref[0] + p_ref[1]
    h2 = jnp.maximum(s * nd[:, None], 0.0)
    h0 = jnp.dot(h2, w_ref[...], preferred_element_type=jnp.float32)
    hcat_ref[...] = jnp.concatenate([h0 * ns[:, None], h0], axis=-1)


def _mid(p1, degp, W_vsgc):
    return pl.pallas_call(
        _mid_body,
        grid=(GRID,),
        in_specs=[
            pl.BlockSpec((NC, BR, D_HID), lambda i: (0, i, 0)),
            pl.BlockSpec((NC, 2, BR), lambda i: (0, 0, i)),
            pl.BlockSpec((D_HID, D_OUT), lambda i: (0, 0)),
        ],
        out_specs=pl.BlockSpec((BR, D_HID), lambda i: (i, 0)),
        out_shape=jax.ShapeDtypeStruct((N, D_HID), jnp.float32),
    )(p1, degp, W_vsgc)


def _fin_body(p_ref, hcat_ref, deg_ref, o_ref):
    nd = lax.rsqrt(jnp.maximum(deg_ref[0, 1] + deg_ref[1, 1], 1.0))
    t = (p_ref[0] + p_ref[1])[:, :D_OUT]
    h0 = hcat_ref[:, D_OUT:]
    o_ref[...] = (h0 + t * nd[:, None]) * 0.5


def _fin(p2, hcat, degp):
    return pl.pallas_call(
        _fin_body,
        grid=(GRID,),
        in_specs=[
            pl.BlockSpec((NC, BR, D_HID), lambda i: (0, i, 0)),
            pl.BlockSpec((BR, D_HID), lambda i: (i, 0)),
            pl.BlockSpec((NC, 2, BR), lambda i: (0, 0, i)),
        ],
        out_specs=pl.BlockSpec((BR, D_OUT), lambda i: (i, 0)),
        out_shape=jax.ShapeDtypeStruct((N, D_OUT), jnp.float32),
    )(p2, hcat, degp)


def kernel(x, edge_index, W_gcn, W_vsgc):
    src_p = edge_index[0].reshape(NW, IB, NBB, CHP)
    dst_p = edge_index[1].reshape(NW, IB, NBB, CHP)
    degp = _degrees(src_p, dst_p)
    hs = _mm1(x, W_gcn, degp)
    p1 = _prop128(src_p, dst_p, hs)
    hcat = _mid(p1, degp, W_vsgc)
    p2 = _prop64(src_p, dst_p, hcat)
    return _fin(p2, hcat, degp)


# speedup vs baseline: 1.0370x; 1.0370x over previous
import functools

import jax
import jax.numpy as jnp
from jax import lax
from jax.experimental import pallas as pl
from jax.experimental.pallas import tpu as pltpu
from jax.experimental.pallas import tpu_sc as plsc

N = 10000
NPAD = 10240
E = 320000
D_IN = 128
D_HID = 128
D_OUT = 64
NC = 2
NS = 16
NW = NC * NS
EW = E // NW
CH = 80
NB = EW // CH
CHP = 40
NBP = EW // CHP
NBB = 25
IB = NBP // NBB
RPT = NPAD // NS
BR = 1024
GRID = NPAD // BR

_mesh = plsc.VectorSubcoreMesh(core_axis_name="c", subcore_axis_name="s")


@functools.partial(
    pl.kernel,
    out_type=jax.ShapeDtypeStruct((NC, 2, NPAD), jnp.float32),
    mesh=_mesh,
    scratch_types=[
        pltpu.VMEM((IB, NBB, CHP), jnp.int32),
        pltpu.VMEM((IB, NBB, CHP), jnp.int32),
        pltpu.VMEM((CHP,), jnp.float32),
        pltpu.VMEM((RPT,), jnp.float32),
        pltpu.VMEM_SHARED((NPAD,), jnp.float32),
        pltpu.VMEM_SHARED((NPAD,), jnp.float32),
        pltpu.SemaphoreType.DMA,
        pltpu.SemaphoreType.DMA,
        pltpu.SemaphoreType.DMA,
        pltpu.SemaphoreType.DMA,
    ],
)
def _degrees(src_hbm, dst_hbm, out_hbm, sidx, didx, ones_v, zer_v, acc_s, acc_d,
             ssem, dsem, is0, is1):
    c = lax.axis_index("c")
    s = lax.axis_index("s")
    wid = c * NS + s
    sd = pltpu.async_copy(src_hbm.at[wid], sidx, is0)
    dd = pltpu.async_copy(dst_hbm.at[wid], didx, is1)
    for i in range(CHP // 16):
        ones_v[pl.ds(i * 16, 16)] = jnp.ones((16,), jnp.float32)
    if CHP % 16:
        ones_v[pl.ds(CHP - 16, 16)] = jnp.ones((16,), jnp.float32)
    for i in range(RPT // 16):
        zer_v[pl.ds(i * 16, 16)] = jnp.zeros((16,), jnp.float32)
    r0 = pl.multiple_of(s * RPT, 128)
    pltpu.sync_copy(zer_v, acc_s.at[pl.ds(r0, RPT)])
    pltpu.sync_copy(zer_v, acc_d.at[pl.ds(r0, RPT)])
    sd.wait()
    dd.wait()
    plsc.subcore_barrier()

    def body(g, carry):
        ds_ = []
        for j in range(5):
            b = g * 5 + j
            ib = b // NBB
            bb = b % NBB
            ds_.append(
                pltpu.async_copy(ones_v, acc_s.at[sidx.at[ib, bb]], dsem, add=True)
            )
            ds_.append(
                pltpu.async_copy(ones_v, acc_d.at[didx.at[ib, bb]], ssem, add=True)
            )
        for d in ds_:
            d.wait()
        return carry

    lax.fori_loop(0, NBP // 5, body, 0)
    plsc.subcore_barrier()
    pltpu.sync_copy(acc_s.at[pl.ds(r0, RPT)], out_hbm.at[c, 0, pl.ds(r0, RPT)])
    pltpu.sync_copy(acc_d.at[pl.ds(r0, RPT)], out_hbm.at[c, 1, pl.ds(r0, RPT)])


NBUF = 5


def _make_prop(DA):
    @functools.partial(
        pl.kernel,
        out_type=jax.ShapeDtypeStruct((NC, NPAD, DA), jnp.float32),
        mesh=_mesh,
        scratch_types=[
            pltpu.VMEM((2, NBB, CHP), jnp.int32),
            pltpu.VMEM((2, NBB, CHP), jnp.int32),
            pltpu.VMEM((NBUF, CHP, D_HID), jnp.float32),
            pltpu.VMEM((16, DA), jnp.float32),
            pltpu.VMEM_SHARED((NPAD, DA), jnp.float32),
        ]
        + [pltpu.SemaphoreType.DMA] * (2 * NBUF + 2),
    )
    def _prop(src_hbm, dst_hbm, feat_hbm, out_hbm, sidx, didx, rows_v, z16, acc, *sems):
        gsems = sems[:NBUF]
        ssems = sems[NBUF : 2 * NBUF]
        isems = sems[2 * NBUF :]
        c = lax.axis_index("c")
        s = lax.axis_index("s")
        wid = c * NS + s
        ids = [None, None]
        ids[0] = (
            pltpu.async_copy(src_hbm.at[wid, 0], sidx.at[0], isems[0]),
            pltpu.async_copy(dst_hbm.at[wid, 0], didx.at[0], isems[1]),
        )
        for r in range(16):
            for j in range(DA // 16):
                z16[r, pl.ds(j * 16, 16)] = jnp.zeros((16,), jnp.float32)
        r0 = pl.multiple_of(s * RPT, 128)
        zds = [
            pltpu.async_copy(z16, acc.at[pl.ds(r0 + k * 16, 16)], gsems[k % NBUF])
            for k in range(RPT // 16)
        ]
        for d in zds:
            d.wait()
        plsc.subcore_barrier()

        for blk in range(IB):
            sl = blk % 2
            for d in ids[sl]:
                d.wait()
            if blk + 1 < IB:
                nsl = (blk + 1) % 2
                ids[nsl] = (
                    pltpu.async_copy(src_hbm.at[wid, blk + 1], sidx.at[nsl], isems[0]),
                    pltpu.async_copy(dst_hbm.at[wid, blk + 1], didx.at[nsl], isems[1]),
                )
            pending = [None] * NBUF
            for g in range(NBB // NBUF):
                gds = []
                for j in range(NBUF):
                    if pending[j] is not None:
                        pending[j].wait()
                    gds.append(
                        pltpu.async_copy(
                            feat_hbm.at[sidx.at[sl, g * NBUF + j]],
                            rows_v.at[j],
                            gsems[j],
                        )
                    )
                for j in range(NBUF):
                    gds[j].wait()
                    pending[j] = pltpu.async_copy(
                        rows_v.at[j],
                        acc.at[didx.at[sl, g * NBUF + j]],
                        ssems[j],
                        add=True,
                    )
            for j in range(NBUF):
                pending[j].wait()
        plsc.subcore_barrier()
        pltpu.sync_copy(acc.at[pl.ds(r0, RPT)], out_hbm.at[c, pl.ds(r0, RPT)])

    return _prop


_prop128 = _make_prop(D_HID)
_prop64 = _prop128


def _mm1_body(x_ref, w_ref, deg_ref, o_ref):
    h = jnp.dot(x_ref[...], w_ref[...], preferred_element_type=jnp.float32)
    ns = lax.rsqrt(jnp.maximum(deg_ref[0, 0] + deg_ref[1, 0], 1.0))
    o_ref[...] = h * ns[:, None]


def _mm1(x, W_gcn, degp):
    return pl.pallas_call(
        _mm1_body,
        grid=(GRID,),
        in_specs=[
            pl.BlockSpec((BR, D_IN), lambda i: (i, 0)),
            pl.BlockSpec((D_IN, D_HID), lambda i: (0, 0)),
            pl.BlockSpec((NC, 2, BR), lambda i: (0, 0, i)),
        ],
        out_specs=pl.BlockSpec((BR, D_HID), lambda i: (i, 0)),
        out_shape=jax.ShapeDtypeStruct((N, D_HID), jnp.float32),
    )(x, W_gcn, degp)


def _mid_body(p_ref, deg_ref, w_ref, hcat_ref):
    nd = lax.rsqrt(jnp.maximum(deg_ref[0, 1] + deg_ref[1, 1], 1.0))
    ns = lax.rsqrt(jnp.maximum(deg_ref[0, 0] + deg_ref[1, 0], 1.0))
    s = p_ref[0] + p_ref[1]
    h2 = jnp.maximum(s * nd[:, None], 0.0)
    h0 = jnp.dot(h2, w_ref[...], preferred_element_type=jnp.float32)
    hcat_ref[...] = jnp.concatenate([h0 * ns[:, None], h0], axis=-1)


def _mid(p1, degp, W_vsgc):
    return pl.pallas_call(
        _mid_body,
        grid=(GRID,),
        in_specs=[
            pl.BlockSpec((NC, BR, D_HID), lambda i: (0, i, 0)),
            pl.BlockSpec((NC, 2, BR), lambda i: (0, 0, i)),
            pl.BlockSpec((D_HID, D_OUT), lambda i: (0, 0)),
        ],
        out_specs=pl.BlockSpec((BR, D_HID), lambda i: (i, 0)),
        out_shape=jax.ShapeDtypeStruct((N, D_HID), jnp.float32),
    )(p1, degp, W_vsgc)


def _fin_body(p_ref, hcat_ref, deg_ref, o_ref):
    nd = lax.rsqrt(jnp.maximum(deg_ref[0, 1] + deg_ref[1, 1], 1.0))
    t = (p_ref[0] + p_ref[1])[:, :D_OUT]
    h0 = hcat_ref[:, D_OUT:]
    o_ref[...] = (h0 + t * nd[:, None]) * 0.5


def _fin(p2, hcat, degp):
    return pl.pallas_call(
        _fin_body,
        grid=(GRID,),
        in_specs=[
            pl.BlockSpec((NC, BR, D_HID), lambda i: (0, i, 0)),
            pl.BlockSpec((BR, D_HID), lambda i: (i, 0)),
            pl.BlockSpec((NC, 2, BR), lambda i: (0, 0, i)),
        ],
        out_specs=pl.BlockSpec((BR, D_OUT), lambda i: (i, 0)),
        out_shape=jax.ShapeDtypeStruct((N, D_OUT), jnp.float32),
    )(p2, hcat, degp)


def kernel(x, edge_index, W_gcn, W_vsgc):
    src_p = edge_index[0].reshape(NW, IB, NBB, CHP)
    dst_p = edge_index[1].reshape(NW, IB, NBB, CHP)
    degp = _degrees(src_p, dst_p)
    hs = _mm1(x, W_gcn, degp)
    p1 = _prop128(src_p, dst_p, hs)
    hcat = _mid(p1, degp, W_vsgc)
    p2 = _prop64(src_p, dst_p, hcat)
    return _fin(p2, hcat, degp)


# speedup vs baseline: 1.0383x; 1.0013x over previous
import functools

import jax
import jax.numpy as jnp
from jax import lax
from jax.experimental import pallas as pl
from jax.experimental.pallas import tpu as pltpu
from jax.experimental.pallas import tpu_sc as plsc

N = 10000
NPAD = 10240
E = 320000
D_IN = 128
D_HID = 128
D_OUT = 64
NC = 2
NS = 16
NW = NC * NS
EW = E // NW
CHP = 40
NBP = EW // CHP
NBB = 25
IB = NBP // NBB
RPT = NPAD // NS
BR = 1024
GRID = NPAD // BR

_mesh = plsc.VectorSubcoreMesh(core_axis_name="c", subcore_axis_name="s")


@functools.partial(
    pl.kernel,
    out_type=jax.ShapeDtypeStruct((NC, 2, NPAD), jnp.float32),
    mesh=_mesh,
    scratch_types=[
        pltpu.VMEM((IB, NBB, CHP), jnp.int32),
        pltpu.VMEM((IB, NBB, CHP), jnp.int32),
        pltpu.VMEM((CHP,), jnp.float32),
        pltpu.VMEM((RPT,), jnp.float32),
        pltpu.VMEM_SHARED((NPAD,), jnp.float32),
        pltpu.VMEM_SHARED((NPAD,), jnp.float32),
        pltpu.SemaphoreType.DMA,
        pltpu.SemaphoreType.DMA,
        pltpu.SemaphoreType.DMA,
        pltpu.SemaphoreType.DMA,
    ],
)
def _degrees(src_hbm, dst_hbm, out_hbm, sidx, didx, ones_v, zer_v, acc_s, acc_d,
             ssem, dsem, is0, is1):
    c = lax.axis_index("c")
    s = lax.axis_index("s")
    wid = c * NS + s
    sd = pltpu.async_copy(src_hbm.at[wid], sidx, is0)
    dd = pltpu.async_copy(dst_hbm.at[wid], didx, is1)
    for i in range(CHP // 16):
        ones_v[pl.ds(i * 16, 16)] = jnp.ones((16,), jnp.float32)
    if CHP % 16:
        ones_v[pl.ds(CHP - 16, 16)] = jnp.ones((16,), jnp.float32)
    for i in range(RPT // 16):
        zer_v[pl.ds(i * 16, 16)] = jnp.zeros((16,), jnp.float32)
    r0 = pl.multiple_of(s * RPT, 128)
    pltpu.sync_copy(zer_v, acc_s.at[pl.ds(r0, RPT)])
    pltpu.sync_copy(zer_v, acc_d.at[pl.ds(r0, RPT)])
    sd.wait()
    dd.wait()
    plsc.subcore_barrier()

    def body(g, carry):
        ds_ = []
        for j in range(5):
            b = g * 5 + j
            ib = b // NBB
            bb = b % NBB
            ds_.append(
                pltpu.async_copy(ones_v, acc_s.at[sidx.at[ib, bb]], dsem, add=True)
            )
            ds_.append(
                pltpu.async_copy(ones_v, acc_d.at[didx.at[ib, bb]], ssem, add=True)
            )
        for d in ds_:
            d.wait()
        return carry

    lax.fori_loop(0, NBP // 5, body, 0)
    plsc.subcore_barrier()
    pltpu.sync_copy(acc_s.at[pl.ds(r0, RPT)], out_hbm.at[c, 0, pl.ds(r0, RPT)])
    pltpu.sync_copy(acc_d.at[pl.ds(r0, RPT)], out_hbm.at[c, 1, pl.ds(r0, RPT)])


NBUF = 5


def _make_prop(DA):
    @functools.partial(
        pl.kernel,
        out_type=jax.ShapeDtypeStruct((NC, NPAD, DA), jnp.float32),
        mesh=_mesh,
        scratch_types=[
            pltpu.VMEM((2, NBB, CHP), jnp.int32),
            pltpu.VMEM((2, NBB, CHP), jnp.int32),
            pltpu.VMEM((NBUF, CHP, D_HID), jnp.float32),
            pltpu.VMEM((16, DA), jnp.float32),
            pltpu.VMEM_SHARED((NPAD, DA), jnp.float32),
        ]
        + [pltpu.SemaphoreType.DMA] * (2 * NBUF + 2),
    )
    def _prop(src_hbm, dst_hbm, feat_hbm, out_hbm, sidx, didx, rows_v, z16, acc, *sems):
        gsems = sems[:NBUF]
        ssems = sems[NBUF : 2 * NBUF]
        isems = sems[2 * NBUF :]
        c = lax.axis_index("c")
        s = lax.axis_index("s")
        wid = c * NS + s
        ids = [None, None]
        ids[0] = (
            pltpu.async_copy(src_hbm.at[wid, 0], sidx.at[0], isems[0]),
            pltpu.async_copy(dst_hbm.at[wid, 0], didx.at[0], isems[1]),
        )
        for r in range(16):
            for j in range(DA // 16):
                z16[r, pl.ds(j * 16, 16)] = jnp.zeros((16,), jnp.float32)
        r0 = pl.multiple_of(s * RPT, 128)
        zds = [
            pltpu.async_copy(z16, acc.at[pl.ds(r0 + k * 16, 16)], gsems[k % NBUF])
            for k in range(RPT // 16)
        ]
        for d in zds:
            d.wait()
        plsc.subcore_barrier()

        for blk in range(IB):
            sl = blk % 2
            for d in ids[sl]:
                d.wait()
            if blk + 1 < IB:
                nsl = (blk + 1) % 2
                ids[nsl] = (
                    pltpu.async_copy(src_hbm.at[wid, blk + 1], sidx.at[nsl], isems[0]),
                    pltpu.async_copy(dst_hbm.at[wid, blk + 1], didx.at[nsl], isems[1]),
                )
            pending = [None] * NBUF
            for g in range(NBB // NBUF):
                gds = []
                for j in range(NBUF):
                    if pending[j] is not None:
                        pending[j].wait()
                    gds.append(
                        pltpu.async_copy(
                            feat_hbm.at[sidx.at[sl, g * NBUF + j]],
                            rows_v.at[j],
                            gsems[j],
                        )
                    )
                for j in range(NBUF):
                    gds[j].wait()
                    pending[j] = pltpu.async_copy(
                        rows_v.at[j],
                        acc.at[didx.at[sl, g * NBUF + j]],
                        ssems[j],
                        add=True,
                    )
            for j in range(NBUF):
                pending[j].wait()
        plsc.subcore_barrier()
        pltpu.sync_copy(acc.at[pl.ds(r0, RPT)], out_hbm.at[c, pl.ds(r0, RPT)])

    return _prop


_prop128 = _make_prop(D_HID)
_prop64 = _prop128


def _mm1_body(x_ref, w_ref, deg_ref, o_ref):
    h = jnp.dot(x_ref[...], w_ref[...], preferred_element_type=jnp.float32)
    ns = lax.rsqrt(jnp.maximum(deg_ref[0, 0] + deg_ref[1, 0], 1.0))
    o_ref[...] = h * ns[:, None]


def _mm1(x, W_gcn, degp):
    return pl.pallas_call(
        _mm1_body,
        grid=(GRID,),
        in_specs=[
            pl.BlockSpec((BR, D_IN), lambda i: (i, 0)),
            pl.BlockSpec((D_IN, D_HID), lambda i: (0, 0)),
            pl.BlockSpec((NC, 2, BR), lambda i: (0, 0, i)),
        ],
        out_specs=pl.BlockSpec((BR, D_HID), lambda i: (i, 0)),
        out_shape=jax.ShapeDtypeStruct((N, D_HID), jnp.float32),
    )(x, W_gcn, degp)


def _mid_body(p_ref, deg_ref, w_ref, hcat_ref):
    nd = lax.rsqrt(jnp.maximum(deg_ref[0, 1] + deg_ref[1, 1], 1.0))
    ns = lax.rsqrt(jnp.maximum(deg_ref[0, 0] + deg_ref[1, 0], 1.0))
    s = p_ref[0] + p_ref[1]
    h2 = jnp.maximum(s * nd[:, None], 0.0)
    h0 = jnp.dot(h2, w_ref[...], preferred_element_type=jnp.float32)
    hcat_ref[...] = jnp.concatenate([h0 * ns[:, None], h0], axis=-1)


def _mid(p1, degp, W_vsgc):
    return pl.pallas_call(
        _mid_body,
        grid=(GRID,),
        in_specs=[
            pl.BlockSpec((NC, BR, D_HID), lambda i: (0, i, 0)),
            pl.BlockSpec((NC, 2, BR), lambda i: (0, 0, i)),
            pl.BlockSpec((D_HID, D_OUT), lambda i: (0, 0)),
        ],
        out_specs=pl.BlockSpec((BR, D_HID), lambda i: (i, 0)),
        out_shape=jax.ShapeDtypeStruct((N, D_HID), jnp.float32),
    )(p1, degp, W_vsgc)


def _fin_body(p_ref, hcat_ref, deg_ref, o_ref):
    nd = lax.rsqrt(jnp.maximum(deg_ref[0, 1] + deg_ref[1, 1], 1.0))
    t = (p_ref[0] + p_ref[1])[:, :D_OUT]
    h0 = hcat_ref[:, D_OUT:]
    o_ref[...] = (h0 + t * nd[:, None]) * 0.5


def _fin(p2, hcat, degp):
    return pl.pallas_call(
        _fin_body,
        grid=(GRID,),
        in_specs=[
            pl.BlockSpec((NC, BR, D_HID), lambda i: (0, i, 0)),
            pl.BlockSpec((BR, D_HID), lambda i: (i, 0)),
            pl.BlockSpec((NC, 2, BR), lambda i: (0, 0, i)),
        ],
        out_specs=pl.BlockSpec((BR, D_OUT), lambda i: (i, 0)),
        out_shape=jax.ShapeDtypeStruct((N, D_OUT), jnp.float32),
    )(p2, hcat, degp)


def kernel(x, edge_index, W_gcn, W_vsgc):
    src_p = edge_index[0].reshape(NW, IB, NBB, CHP)
    dst_p = edge_index[1].reshape(NW, IB, NBB, CHP)
    degp = _degrees(src_p, dst_p)
    hs = _mm1(x, W_gcn, degp)
    p1 = _prop128(src_p, dst_p, hs)
    hcat = _mid(p1, degp, W_vsgc)
    p2 = _prop64(src_p, dst_p, hcat)
    return _fin(p2, hcat, degp)
